# Initial kernel scaffold; baseline (speedup 1.0000x reference)
import jax
import jax.numpy as jnp
from jax.experimental import pallas as pl


def kernel(input_ids, type_ids, dpe_ids, times, input_table, type_table, dpe_table, ln_weight, ln_bias):
    raise NotImplementedError("write your pallas kernel here")


# speedup vs baseline: 1.5085x; 1.5085x over previous
import functools

import jax
import jax.numpy as jnp
from jax import lax
from jax.experimental import pallas as pl
from jax.experimental.pallas import tpu as pltpu
from jax.experimental.pallas import tpu_sc as plsc

PRED_DIM = 128
MAX_WORD_LEN = 32
B, S, W = 128, 128, 32
N_TOK = B * S * W

NUM_WORKERS = 32
TPW = N_TOK // NUM_WORKERS
CHUNK = 128
NCHUNK = TPW // CHUNK
NJ = PRED_DIM // 16


def _make_pe_table():
    position = jnp.arange(MAX_WORD_LEN, dtype=jnp.float32)[:, None]
    div_term = jnp.exp(
        jnp.arange(0, PRED_DIM, 2, dtype=jnp.float32)
        * (-jnp.log(10000.0) / PRED_DIM))
    pe = jnp.zeros((MAX_WORD_LEN, PRED_DIM), dtype=jnp.float32)
    pe = pe.at[:, 0::2].set(jnp.sin(position * div_term))
    pe = pe.at[:, 1::2].set(jnp.cos(position * div_term))
    return pe


def _sc_body(it, tt, dt, pe, lnw, lnb, iid, tid, did, out,
             tt_v, dt_v, pe_v, lnw_v, lnb_v,
             iidx_v, tidx_v, didx_v, rows_v, gsem):
    nc = 2
    wid = lax.axis_index("s") * nc + lax.axis_index("c")
    base0 = wid * TPW

    pltpu.sync_copy(tt, tt_v)
    pltpu.sync_copy(dt, dt_v)
    pltpu.sync_copy(pe, pe_v)
    pltpu.sync_copy(lnw, lnw_v)
    pltpu.sync_copy(lnb, lnb_v)

    lanes = lax.iota(jnp.int32, 16)

    def group_body(k, _):
        tvec = tidx_v[pl.ds(16 * k, 16)]
        dvec = didx_v[pl.ds(16 * k, 16)]
        for lane in range(16):
            _one_token(k, lane, tvec, dvec)
        return 0

    def _one_token(k, lane, tvec, dvec):
        t = 16 * k + lane
        tsp = jnp.full((16,), tvec[lane], dtype=jnp.int32)
        dsp = jnp.full((16,), dvec[lane], dtype=jnp.int32)
        w = lax.rem(t, MAX_WORD_LEN)

        xs = []
        for j in range(NJ):
            col = lanes + (16 * j)
            v = rows_v[t, pl.ds(16 * j, 16)]
            v = v + plsc.load_gather(tt_v, [tsp, col])
            v = v + plsc.load_gather(dt_v, [dsp, col])
            v = v + pe_v[w, pl.ds(16 * j, 16)]
            xs.append(v)

        s01 = xs[0] + xs[1]
        s23 = xs[2] + xs[3]
        s45 = xs[4] + xs[5]
        s67 = xs[6] + xs[7]
        tot = (s01 + s23) + (s45 + s67)
        mean_s = jnp.sum(tot) * (1.0 / PRED_DIM)
        m = jnp.full((16,), mean_s)

        ds_ = [x - m for x in xs]
        q01 = ds_[0] * ds_[0] + ds_[1] * ds_[1]
        q23 = ds_[2] * ds_[2] + ds_[3] * ds_[3]
        q45 = ds_[4] * ds_[4] + ds_[5] * ds_[5]
        q67 = ds_[6] * ds_[6] + ds_[7] * ds_[7]
        qtot = (q01 + q23) + (q45 + q67)
        var_s = jnp.sum(qtot) * (1.0 / PRED_DIM)

        vv = jnp.full((16,), var_s + 1e-12)
        bits = plsc.bitcast(vv, jnp.int32)
        bits = jnp.int32(0x5F3759DF) - lax.shift_right_logical(bits, 1)
        y = plsc.bitcast(bits, jnp.float32)
        half_vv = vv * 0.5
        for _ in range(4):
            y = y * (1.5 - half_vv * y * y)

        for j in range(NJ):
            o = (ds_[j] * y) * lnw_v[pl.ds(16 * j, 16)] \
                + lnb_v[pl.ds(16 * j, 16)]
            rows_v[t, pl.ds(16 * j, 16)] = o

    def chunk_body(g, _):
        gb = base0 + g * CHUNK
        pltpu.sync_copy(iid.at[pl.ds(gb, CHUNK)], iidx_v)
        pltpu.sync_copy(tid.at[pl.ds(gb, CHUNK)], tidx_v)
        pltpu.sync_copy(did.at[pl.ds(gb, CHUNK)], didx_v)
        pltpu.async_copy(it.at[iidx_v], rows_v, gsem).wait()
        lax.fori_loop(0, CHUNK // 16, group_body, 0)
        pltpu.sync_copy(rows_v, out.at[pl.ds(gb, CHUNK)])
        return 0

    lax.fori_loop(0, NCHUNK, chunk_body, 0)


@functools.partial(
    pl.kernel,
    out_type=jax.ShapeDtypeStruct((N_TOK, PRED_DIM), jnp.float32),
    mesh=plsc.VectorSubcoreMesh(core_axis_name="c", subcore_axis_name="s"),
    compiler_params=pltpu.CompilerParams(needs_layout_passes=False),
    scratch_types=[
        pltpu.VMEM((8, PRED_DIM), jnp.float32),
        pltpu.VMEM((100, PRED_DIM), jnp.float32),
        pltpu.VMEM((MAX_WORD_LEN, PRED_DIM), jnp.float32),
        pltpu.VMEM((PRED_DIM,), jnp.float32),
        pltpu.VMEM((PRED_DIM,), jnp.float32),
        pltpu.VMEM((CHUNK,), jnp.int32),
        pltpu.VMEM((CHUNK,), jnp.int32),
        pltpu.VMEM((CHUNK,), jnp.int32),
        pltpu.VMEM((CHUNK, PRED_DIM), jnp.float32),
        pltpu.SemaphoreType.DMA,
    ],
)
def _desc_emb_sc(it, tt, dt, pe, lnw, lnb, iid, tid, did, out, *scratch):
    _sc_body(it, tt, dt, pe, lnw, lnb, iid, tid, did, out, *scratch)


def kernel(input_ids, type_ids, dpe_ids, times, input_table, type_table,
           dpe_table, ln_weight, ln_bias):
    del times
    it = input_table.at[0].set(0.0)
    tt = type_table.at[0].set(0.0)
    dt = dpe_table.at[0].set(0.0)
    pe = _make_pe_table()
    iid = input_ids.reshape(-1).astype(jnp.int32)
    tid = type_ids.reshape(-1).astype(jnp.int32)
    did = dpe_ids.reshape(-1).astype(jnp.int32)
    out = _desc_emb_sc(it, tt, dt, pe, ln_weight, ln_bias, iid, tid, did)
    return out.reshape(B * S, W, PRED_DIM)
